# speedup vs baseline: 125.2247x; 125.2247x over previous
import functools

import jax
import jax.numpy as jnp
from jax import lax
from jax.experimental import pallas as pl
from jax.experimental.pallas import tpu as pltpu, tpu_sc as plsc

L = 16
NS = 16


def _sc_graph_step(vals_pad, src, dst, w, *, npad, e):
    ept = e // NS
    chunk = 8000
    assert ept % chunk == 0
    nchunks = ept // chunk
    slc = npad // NS
    assert slc % L == 0

    mesh = plsc.VectorSubcoreMesh(
        core_axis_name="c", subcore_axis_name="s", num_cores=1)

    @functools.partial(
        pl.kernel,
        out_type=[
            jax.ShapeDtypeStruct((npad,), jnp.float32),
            jax.ShapeDtypeStruct((npad,), jnp.float32),
            jax.ShapeDtypeStruct((npad,), jnp.float32),
        ],
        mesh=mesh,
        compiler_params=pltpu.CompilerParams(needs_layout_passes=False),
        scratch_types=[
            pltpu.VMEM((npad,), jnp.float32),
            pltpu.VMEM((npad,), jnp.float32),
            pltpu.VMEM((chunk,), jnp.int32),
            pltpu.VMEM((chunk,), jnp.int32),
            pltpu.VMEM((chunk,), jnp.float32),
            pltpu.VMEM((NS * slc,), jnp.float32),
            pltpu.VMEM((slc,), jnp.float32),
            pltpu.VMEM((slc,), jnp.float32),
            pltpu.VMEM((slc,), jnp.float32),
            pltpu.VMEM((slc,), jnp.float32),
            pltpu.VMEM_SHARED((npad,), jnp.float32),
            pltpu.VMEM_SHARED((NS * npad,), jnp.float32),
        ],
    )
    def body(vals_hbm, src_hbm, dst_hbm, w_hbm,
             preds_hbm, err_hbm, delta_hbm,
             tab_v, acc_v, src_v, dst_v, w_v, red_v,
             vals_s, fx_s, err_s, sum_s, tab_sh, part_sh):
        s = lax.axis_index("s")
        base = s * slc
        ebase = s * ept

        pltpu.sync_copy(vals_hbm.at[pl.ds(base, slc)], vals_s)
        for i in range(slc // L):
            v = vals_s[pl.ds(i * L, L)]
            fx_s[pl.ds(i * L, L)] = 1.0 - 2.0 / (jnp.exp(2.0 * v) + 1.0)
        pltpu.sync_copy(fx_s, tab_sh.at[pl.ds(base, slc)])
        plsc.subcore_barrier()
        pltpu.sync_copy(tab_sh, tab_v)

        def zero_acc():
            @pl.loop(0, npad, step=L)
            def _(i):
                acc_v[pl.ds(i, L)] = jnp.zeros((L,), jnp.float32)

        def edge_pass(gather_first):
            for c in range(nchunks):
                eoff = ebase + c * chunk
                pltpu.sync_copy(src_hbm.at[pl.ds(eoff, chunk)], src_v)
                pltpu.sync_copy(dst_hbm.at[pl.ds(eoff, chunk)], dst_v)
                pltpu.sync_copy(w_hbm.at[pl.ds(eoff, chunk)], w_v)

                @pl.loop(0, chunk, step=L)
                def _(i):
                    sv = src_v[pl.ds(i, L)]
                    dv = dst_v[pl.ds(i, L)]
                    wv = w_v[pl.ds(i, L)]
                    gidx = sv if gather_first else dv
                    sidx = dv if gather_first else sv
                    g = plsc.load_gather(tab_v, [gidx])
                    plsc.addupdate_scatter(acc_v, [sidx], wv * g)

        def reduce_partials(out_s):
            pltpu.sync_copy(acc_v, part_sh.at[pl.ds(s * npad, npad)])
            plsc.subcore_barrier()
            for j in range(NS):
                pltpu.sync_copy(part_sh.at[pl.ds(j * npad + base, slc)],
                                red_v.at[pl.ds(j * slc, slc)])

            @pl.loop(0, slc, step=L)
            def _(i):
                t = red_v[pl.ds(i, L)]
                for j in range(1, NS):
                    t = t + red_v[pl.ds(j * slc + i, L)]
                out_s[pl.ds(i, L)] = t

        zero_acc()
        edge_pass(gather_first=True)
        reduce_partials(sum_s)

        for i in range(slc // L):
            err_s[pl.ds(i * L, L)] = vals_s[pl.ds(i * L, L)] - sum_s[pl.ds(i * L, L)]
        pltpu.sync_copy(sum_s, preds_hbm.at[pl.ds(base, slc)])
        pltpu.sync_copy(err_s, err_hbm.at[pl.ds(base, slc)])
        pltpu.sync_copy(err_s, tab_sh.at[pl.ds(base, slc)])
        plsc.subcore_barrier()
        pltpu.sync_copy(tab_sh, tab_v)

        zero_acc()
        edge_pass(gather_first=False)
        reduce_partials(sum_s)

        for i in range(slc // L):
            fx = fx_s[pl.ds(i * L, L)]
            err_s[pl.ds(i * L, L)] = (1.0 - fx * fx) * sum_s[pl.ds(i * L, L)] - err_s[pl.ds(i * L, L)]
        pltpu.sync_copy(err_s, delta_hbm.at[pl.ds(base, slc)])

    return body(vals_pad, src, dst, w)


def kernel(x, edge_index, weights):
    n = x.shape[0]
    e = edge_index.shape[1]
    npad = ((n + NS * L - 1) // (NS * L)) * (NS * L)
    vals = x[:, 0]
    vals_pad = jnp.zeros((npad,), jnp.float32).at[:n].set(vals)
    preds, errors, delta = _sc_graph_step(
        vals_pad, edge_index[0], edge_index[1], weights, npad=npad, e=e)
    return jnp.stack([preds[:n], errors[:n], delta[:n]], axis=1)


# speedup vs baseline: 131.7618x; 1.0522x over previous
import functools

import jax
import jax.numpy as jnp
from jax import lax
from jax.experimental import pallas as pl
from jax.experimental.pallas import tpu as pltpu, tpu_sc as plsc

L = 16
NS = 16


def _sc_graph_step(vals_pad, src, dst, w, *, npad, e):
    ept = e // NS
    chunk = 8000
    assert ept % chunk == 0
    nchunks = ept // chunk
    slc = npad // NS
    assert slc % L == 0

    mesh = plsc.VectorSubcoreMesh(
        core_axis_name="c", subcore_axis_name="s", num_cores=1)

    @functools.partial(
        pl.kernel,
        out_type=[
            jax.ShapeDtypeStruct((npad,), jnp.float32),
            jax.ShapeDtypeStruct((npad,), jnp.float32),
            jax.ShapeDtypeStruct((npad,), jnp.float32),
        ],
        mesh=mesh,
        compiler_params=pltpu.CompilerParams(needs_layout_passes=False),
        scratch_types=[
            pltpu.VMEM((npad,), jnp.float32),
            pltpu.VMEM((npad,), jnp.float32),
            pltpu.VMEM((chunk,), jnp.int32),
            pltpu.VMEM((chunk,), jnp.int32),
            pltpu.VMEM((chunk,), jnp.float32),
            pltpu.VMEM((NS * slc,), jnp.float32),
            pltpu.VMEM((slc,), jnp.float32),
            pltpu.VMEM((slc,), jnp.float32),
            pltpu.VMEM((slc,), jnp.float32),
            pltpu.VMEM((slc,), jnp.float32),
            pltpu.VMEM_SHARED((npad,), jnp.float32),
            pltpu.VMEM_SHARED((NS * npad,), jnp.float32),
        ],
    )
    def body(vals_hbm, src_hbm, dst_hbm, w_hbm,
             preds_hbm, err_hbm, delta_hbm,
             tab_v, acc_v, src_v, dst_v, w_v, red_v,
             vals_s, fx_s, err_s, sum_s, tab_sh, part_sh):
        s = lax.axis_index("s")
        base = s * slc
        ebase = s * ept

        pltpu.sync_copy(vals_hbm.at[pl.ds(base, slc)], vals_s)
        for i in range(slc // L):
            v = vals_s[pl.ds(i * L, L)]
            fx_s[pl.ds(i * L, L)] = 1.0 - 2.0 / (jnp.exp(2.0 * v) + 1.0)
        pltpu.sync_copy(fx_s, tab_sh.at[pl.ds(base, slc)])
        plsc.subcore_barrier()
        pltpu.sync_copy(tab_sh, tab_v)

        def zero_acc():
            @pl.loop(0, npad, step=L, unroll=8)
            def _(i):
                acc_v[pl.ds(i, L)] = jnp.zeros((L,), jnp.float32)

        def edge_pass(gather_first):
            for c in range(nchunks):
                eoff = ebase + c * chunk
                pltpu.sync_copy(src_hbm.at[pl.ds(eoff, chunk)], src_v)
                pltpu.sync_copy(dst_hbm.at[pl.ds(eoff, chunk)], dst_v)
                pltpu.sync_copy(w_hbm.at[pl.ds(eoff, chunk)], w_v)

                @pl.loop(0, chunk, step=L, unroll=8)
                def _(i):
                    sv = src_v[pl.ds(i, L)]
                    dv = dst_v[pl.ds(i, L)]
                    wv = w_v[pl.ds(i, L)]
                    gidx = sv if gather_first else dv
                    sidx = dv if gather_first else sv
                    g = plsc.load_gather(tab_v, [gidx])
                    plsc.addupdate_scatter(acc_v, [sidx], wv * g)

        def reduce_partials(out_s):
            pltpu.sync_copy(acc_v, part_sh.at[pl.ds(s * npad, npad)])
            plsc.subcore_barrier()
            for j in range(NS):
                pltpu.sync_copy(part_sh.at[pl.ds(j * npad + base, slc)],
                                red_v.at[pl.ds(j * slc, slc)])

            @pl.loop(0, slc, step=L, unroll=2)
            def _(i):
                t = red_v[pl.ds(i, L)]
                for j in range(1, NS):
                    t = t + red_v[pl.ds(j * slc + i, L)]
                out_s[pl.ds(i, L)] = t

        zero_acc()
        edge_pass(gather_first=True)
        reduce_partials(sum_s)

        for i in range(slc // L):
            err_s[pl.ds(i * L, L)] = vals_s[pl.ds(i * L, L)] - sum_s[pl.ds(i * L, L)]
        pltpu.sync_copy(sum_s, preds_hbm.at[pl.ds(base, slc)])
        pltpu.sync_copy(err_s, err_hbm.at[pl.ds(base, slc)])
        pltpu.sync_copy(err_s, tab_sh.at[pl.ds(base, slc)])
        plsc.subcore_barrier()
        pltpu.sync_copy(tab_sh, tab_v)

        zero_acc()
        edge_pass(gather_first=False)
        reduce_partials(sum_s)

        for i in range(slc // L):
            fx = fx_s[pl.ds(i * L, L)]
            err_s[pl.ds(i * L, L)] = (1.0 - fx * fx) * sum_s[pl.ds(i * L, L)] - err_s[pl.ds(i * L, L)]
        pltpu.sync_copy(err_s, delta_hbm.at[pl.ds(base, slc)])

    return body(vals_pad, src, dst, w)


def kernel(x, edge_index, weights):
    n = x.shape[0]
    e = edge_index.shape[1]
    npad = ((n + NS * L - 1) // (NS * L)) * (NS * L)
    vals = x[:, 0]
    vals_pad = jnp.zeros((npad,), jnp.float32).at[:n].set(vals)
    preds, errors, delta = _sc_graph_step(
        vals_pad, edge_index[0], edge_index[1], weights, npad=npad, e=e)
    return jnp.stack([preds[:n], errors[:n], delta[:n]], axis=1)


# speedup vs baseline: 180.4668x; 1.3696x over previous
import functools

import jax
import jax.numpy as jnp
from jax import lax
from jax.experimental import pallas as pl
from jax.experimental.pallas import tpu as pltpu, tpu_sc as plsc

L = 16
NS = 16


def _sc_graph_step(vals_pad, src, dst, w, *, npad, e):
    ept = e // NS
    chunk = 8000
    assert ept % chunk == 0
    nchunks = ept // chunk
    slc = npad // NS
    assert slc % L == 0

    mesh = plsc.VectorSubcoreMesh(
        core_axis_name="c", subcore_axis_name="s", num_cores=1)

    @functools.partial(
        pl.kernel,
        out_type=[
            jax.ShapeDtypeStruct((npad,), jnp.float32),
            jax.ShapeDtypeStruct((npad,), jnp.float32),
            jax.ShapeDtypeStruct((npad,), jnp.float32),
        ],
        mesh=mesh,
        compiler_params=pltpu.CompilerParams(needs_layout_passes=False),
        scratch_types=[
            pltpu.VMEM((npad,), jnp.float32),
            pltpu.VMEM((npad,), jnp.float32),
            pltpu.VMEM((chunk,), jnp.int32),
            pltpu.VMEM((chunk,), jnp.int32),
            pltpu.VMEM((chunk,), jnp.float32),
            pltpu.VMEM((NS * slc,), jnp.float32),
            pltpu.VMEM((slc,), jnp.float32),
            pltpu.VMEM((slc,), jnp.float32),
            pltpu.VMEM((slc,), jnp.float32),
            pltpu.VMEM((slc,), jnp.float32),
            pltpu.VMEM_SHARED((npad,), jnp.float32),
            pltpu.VMEM_SHARED((NS * npad,), jnp.float32),
        ],
    )
    def body(vals_hbm, src_hbm, dst_hbm, w_hbm,
             preds_hbm, err_hbm, delta_hbm,
             tab_v, acc_v, src_v, dst_v, w_v, red_v,
             vals_s, fx_s, err_s, sum_s, tab_sh, part_sh):
        s = lax.axis_index("s")
        base = s * slc
        ebase = s * ept

        pltpu.sync_copy(vals_hbm.at[pl.ds(base, slc)], vals_s)
        for i in range(slc // L):
            v = vals_s[pl.ds(i * L, L)]
            fx_s[pl.ds(i * L, L)] = 1.0 - 2.0 / (jnp.exp(2.0 * v) + 1.0)
        pltpu.sync_copy(fx_s, tab_sh.at[pl.ds(base, slc)])
        plsc.subcore_barrier()
        pltpu.sync_copy(tab_sh, tab_v)

        def zero_acc():
            @pl.loop(0, npad, step=L, unroll=8)
            def _(i):
                acc_v[pl.ds(i, L)] = jnp.zeros((L,), jnp.float32)

        def edge_pass(gather_first):
            for c in range(nchunks):
                eoff = ebase + c * chunk
                pltpu.sync_copy(src_hbm.at[pl.ds(eoff, chunk)], src_v)
                pltpu.sync_copy(dst_hbm.at[pl.ds(eoff, chunk)], dst_v)
                pltpu.sync_copy(w_hbm.at[pl.ds(eoff, chunk)], w_v)

                @plsc.parallel_loop(0, chunk, L, unroll=8)
                def _(i):
                    sv = src_v[pl.ds(i, L)]
                    dv = dst_v[pl.ds(i, L)]
                    wv = w_v[pl.ds(i, L)]
                    gidx = sv if gather_first else dv
                    sidx = dv if gather_first else sv
                    g = plsc.load_gather(tab_v, [gidx])
                    plsc.addupdate_scatter(acc_v, [sidx], wv * g)

        def reduce_partials(out_s):
            pltpu.sync_copy(acc_v, part_sh.at[pl.ds(s * npad, npad)])
            plsc.subcore_barrier()
            for j in range(NS):
                pltpu.sync_copy(part_sh.at[pl.ds(j * npad + base, slc)],
                                red_v.at[pl.ds(j * slc, slc)])

            @pl.loop(0, slc, step=L, unroll=2)
            def _(i):
                t = red_v[pl.ds(i, L)]
                for j in range(1, NS):
                    t = t + red_v[pl.ds(j * slc + i, L)]
                out_s[pl.ds(i, L)] = t

        zero_acc()
        edge_pass(gather_first=True)
        reduce_partials(sum_s)

        for i in range(slc // L):
            err_s[pl.ds(i * L, L)] = vals_s[pl.ds(i * L, L)] - sum_s[pl.ds(i * L, L)]
        pltpu.sync_copy(sum_s, preds_hbm.at[pl.ds(base, slc)])
        pltpu.sync_copy(err_s, err_hbm.at[pl.ds(base, slc)])
        pltpu.sync_copy(err_s, tab_sh.at[pl.ds(base, slc)])
        plsc.subcore_barrier()
        pltpu.sync_copy(tab_sh, tab_v)

        zero_acc()
        edge_pass(gather_first=False)
        reduce_partials(sum_s)

        for i in range(slc // L):
            fx = fx_s[pl.ds(i * L, L)]
            err_s[pl.ds(i * L, L)] = (1.0 - fx * fx) * sum_s[pl.ds(i * L, L)] - err_s[pl.ds(i * L, L)]
        pltpu.sync_copy(err_s, delta_hbm.at[pl.ds(base, slc)])

    return body(vals_pad, src, dst, w)


def kernel(x, edge_index, weights):
    n = x.shape[0]
    e = edge_index.shape[1]
    npad = ((n + NS * L - 1) // (NS * L)) * (NS * L)
    vals = x[:, 0]
    vals_pad = jnp.zeros((npad,), jnp.float32).at[:n].set(vals)
    preds, errors, delta = _sc_graph_step(
        vals_pad, edge_index[0], edge_index[1], weights, npad=npad, e=e)
    return jnp.stack([preds[:n], errors[:n], delta[:n]], axis=1)


# speedup vs baseline: 244.0730x; 1.3525x over previous
import functools

import jax
import jax.numpy as jnp
from jax import lax
from jax.experimental import pallas as pl
from jax.experimental.pallas import tpu as pltpu, tpu_sc as plsc

L = 16
NS = 16


def _sc_graph_step(vals_pad, src, dst, w, *, npad, e):
    ept = e // NS
    chunk = 8000
    assert ept % chunk == 0
    nchunks = ept // chunk
    slc = npad // NS
    assert slc % L == 0

    mesh = plsc.VectorSubcoreMesh(
        core_axis_name="c", subcore_axis_name="s", num_cores=1)

    @functools.partial(
        pl.kernel,
        out_type=[
            jax.ShapeDtypeStruct((npad,), jnp.float32),
            jax.ShapeDtypeStruct((npad,), jnp.float32),
            jax.ShapeDtypeStruct((npad,), jnp.float32),
        ],
        mesh=mesh,
        compiler_params=pltpu.CompilerParams(needs_layout_passes=False),
        scratch_types=[
            pltpu.VMEM((npad,), jnp.float32),
            pltpu.VMEM((npad,), jnp.float32),
            pltpu.VMEM((chunk,), jnp.int32),
            pltpu.VMEM((chunk,), jnp.int32),
            pltpu.VMEM((chunk,), jnp.float32),
            pltpu.VMEM((chunk,), jnp.int32),
            pltpu.VMEM((chunk,), jnp.int32),
            pltpu.VMEM((chunk,), jnp.float32),
            pltpu.SemaphoreType.DMA,
            pltpu.SemaphoreType.DMA,
            pltpu.VMEM((NS * slc,), jnp.float32),
            pltpu.VMEM((slc,), jnp.float32),
            pltpu.VMEM((slc,), jnp.float32),
            pltpu.VMEM((slc,), jnp.float32),
            pltpu.VMEM((slc,), jnp.float32),
            pltpu.VMEM_SHARED((npad,), jnp.float32),
            pltpu.VMEM_SHARED((NS * npad,), jnp.float32),
        ],
    )
    def body(vals_hbm, src_hbm, dst_hbm, w_hbm,
             preds_hbm, err_hbm, delta_hbm,
             tab_v, acc_v, src_v0, dst_v0, w_v0, src_v1, dst_v1, w_v1,
             esem0, esem1, red_v,
             vals_s, fx_s, err_s, sum_s, tab_sh, part_sh):
        ebufs = ((src_v0, dst_v0, w_v0), (src_v1, dst_v1, w_v1))
        esems = (esem0, esem1)
        s = lax.axis_index("s")
        base = s * slc
        ebase = s * ept

        pltpu.sync_copy(vals_hbm.at[pl.ds(base, slc)], vals_s)
        for i in range(slc // L):
            v = vals_s[pl.ds(i * L, L)]
            fx_s[pl.ds(i * L, L)] = 1.0 - 2.0 / (jnp.exp(2.0 * v) + 1.0)
        pltpu.sync_copy(fx_s, tab_sh.at[pl.ds(base, slc)])
        plsc.subcore_barrier()
        pltpu.sync_copy(tab_sh, tab_v)

        def zero_acc():
            @pl.loop(0, npad, step=L, unroll=8)
            def _(i):
                acc_v[pl.ds(i, L)] = jnp.zeros((L,), jnp.float32)

        def start_chunk(c):
            b = c % 2
            eoff = ebase + c * chunk
            return [
                pltpu.async_copy(hbm.at[pl.ds(eoff, chunk)], v, esems[b])
                for hbm, v in zip((src_hbm, dst_hbm, w_hbm), ebufs[b])
            ]

        def edge_pass(gather_first):
            pending = {0: start_chunk(0)}
            for c in range(nchunks):
                if c + 1 < nchunks:
                    pending[c + 1] = start_chunk(c + 1)
                for cp in pending.pop(c):
                    cp.wait()
                src_v, dst_v, w_v = ebufs[c % 2]

                @plsc.parallel_loop(0, chunk, L, unroll=8)
                def _(i):
                    sv = src_v[pl.ds(i, L)]
                    dv = dst_v[pl.ds(i, L)]
                    wv = w_v[pl.ds(i, L)]
                    gidx = sv if gather_first else dv
                    sidx = dv if gather_first else sv
                    g = plsc.load_gather(tab_v, [gidx])
                    plsc.addupdate_scatter(acc_v, [sidx], wv * g)

        def reduce_partials(out_s):
            pltpu.sync_copy(acc_v, part_sh.at[pl.ds(s * npad, npad)])
            plsc.subcore_barrier()
            for j in range(NS):
                pltpu.sync_copy(part_sh.at[pl.ds(j * npad + base, slc)],
                                red_v.at[pl.ds(j * slc, slc)])

            @pl.loop(0, slc, step=L, unroll=2)
            def _(i):
                t = red_v[pl.ds(i, L)]
                for j in range(1, NS):
                    t = t + red_v[pl.ds(j * slc + i, L)]
                out_s[pl.ds(i, L)] = t

        zero_acc()
        edge_pass(gather_first=True)
        reduce_partials(sum_s)

        for i in range(slc // L):
            err_s[pl.ds(i * L, L)] = vals_s[pl.ds(i * L, L)] - sum_s[pl.ds(i * L, L)]
        pltpu.sync_copy(sum_s, preds_hbm.at[pl.ds(base, slc)])
        pltpu.sync_copy(err_s, err_hbm.at[pl.ds(base, slc)])
        pltpu.sync_copy(err_s, tab_sh.at[pl.ds(base, slc)])
        plsc.subcore_barrier()
        pltpu.sync_copy(tab_sh, tab_v)

        zero_acc()
        edge_pass(gather_first=False)
        reduce_partials(sum_s)

        for i in range(slc // L):
            fx = fx_s[pl.ds(i * L, L)]
            err_s[pl.ds(i * L, L)] = (1.0 - fx * fx) * sum_s[pl.ds(i * L, L)] - err_s[pl.ds(i * L, L)]
        pltpu.sync_copy(err_s, delta_hbm.at[pl.ds(base, slc)])

    return body(vals_pad, src, dst, w)


def kernel(x, edge_index, weights):
    n = x.shape[0]
    e = edge_index.shape[1]
    npad = ((n + NS * L - 1) // (NS * L)) * (NS * L)
    vals = x[:, 0]
    vals_pad = jnp.zeros((npad,), jnp.float32).at[:n].set(vals)
    preds, errors, delta = _sc_graph_step(
        vals_pad, edge_index[0], edge_index[1], weights, npad=npad, e=e)
    return jnp.stack([preds[:n], errors[:n], delta[:n]], axis=1)
